# Initial kernel scaffold; baseline (speedup 1.0000x reference)
import jax
import jax.numpy as jnp
from jax.experimental import pallas as pl


def kernel(x, edge_index, batch, c1_W1, c1_b1, c1_g, c1_be, c1_W2, c1_b2, c2_W1, c2_b1, c2_g, c2_be, c2_W2, c2_b2, c3_W1, c3_b1, c3_g, c3_be, c3_W2, c3_b2, lin1_W, lin1_b, lin2_W, lin2_b):
    raise NotImplementedError("write your pallas kernel here")


# speedup vs baseline: 15.8552x; 15.8552x over previous
import functools

import jax
import jax.numpy as jnp
from jax import lax
from jax.experimental import pallas as pl
from jax.experimental.pallas import tpu as pltpu
from jax.experimental.pallas import tpu_sc as plsc

_N = 100000
_E = 6400000
_DH = 64
_NG = 1024

_K = 1600
_NSC = 2
_NTILE = 16


def _tile_stripe(s, n):
  zr = -(-(n // _NTILE) // 8) * 8
  start = jnp.minimum(s * zr, n - zr)
  return pl.multiple_of(start, 8), zr


_SC_PARAMS = pltpu.CompilerParams(use_tc_tiling_on_sc=False)


def _agg_first_layer(table, src, dst, zeros):
  n = table.shape[0]
  e = src.shape[0]
  ept = e // (_NSC * _NTILE)
  nblk = ept // _K

  mesh = plsc.VectorSubcoreMesh(core_axis_name="c", subcore_axis_name="s")

  @functools.partial(
      pl.kernel, mesh=mesh,
      out_type=jax.ShapeDtypeStruct((n, 2 * 16), jnp.float32),
      scratch_types=[
          pltpu.VMEM((_K,), jnp.int32),
          pltpu.VMEM((_K,), jnp.int32),
          pltpu.VMEM((_K, 16), jnp.float32),
          pltpu.VMEM_SHARED((n, 16), jnp.float32),
          pltpu.SemaphoreType.DMA,
      ],
      compiler_params=_SC_PARAMS,
  )
  def kern(table_hbm, src_hbm, dst_hbm, zeros_hbm, out_hbm,
           sidx, didx, rows, acc, sem):
    c = lax.axis_index("c")
    s = lax.axis_index("s")
    r0, zr = _tile_stripe(s, n)
    pltpu.sync_copy(zeros_hbm.at[pl.ds(r0, zr)], acc.at[pl.ds(r0, zr)])
    plsc.subcore_barrier()
    tile_base = (c * _NTILE + s) * ept

    def blk(b, carry):
      e0 = pl.multiple_of(tile_base + b * _K, 8)
      pltpu.sync_copy(src_hbm.at[pl.ds(e0, _K)], sidx)
      pltpu.sync_copy(dst_hbm.at[pl.ds(e0, _K)], didx)
      pltpu.async_copy(table_hbm.at[sidx], rows, sem).wait()
      pltpu.sync_copy(rows, acc.at[didx], add=True)
      return carry

    lax.fori_loop(0, nblk, blk, 0)
    plsc.subcore_barrier()
    lane0 = pl.multiple_of(c * 16, 16)
    pltpu.sync_copy(acc.at[pl.ds(r0, zr)],
                    out_hbm.at[pl.ds(r0, zr), pl.ds(lane0, 16)])

  return kern(table, src, dst, zeros)


def _agg_grouped(table, src, dst, zeros):
  n = table.shape[0] // 4
  e = src.shape[0]
  ept = e // _NTILE
  nblk = ept // _K

  mesh = plsc.VectorSubcoreMesh(core_axis_name="c", subcore_axis_name="s")

  @functools.partial(
      pl.kernel, mesh=mesh,
      out_type=jax.ShapeDtypeStruct((n, 64), jnp.float32),
      scratch_types=[
          pltpu.VMEM((_K,), jnp.int32),
          pltpu.VMEM((_K,), jnp.int32),
          pltpu.VMEM((_K,), jnp.int32),
          pltpu.VMEM((_K, 16), jnp.float32),
          pltpu.VMEM_SHARED((n, 16), jnp.float32),
          pltpu.SemaphoreType.DMA,
      ],
      compiler_params=_SC_PARAMS,
  )
  def kern(table_hbm, src_hbm, dst_hbm, zeros_hbm, out_hbm,
           sidx, didx, gidx, rows, acc, sem):
    c = lax.axis_index("c")
    s = lax.axis_index("s")
    r0, zr = _tile_stripe(s, n)
    tile_base = s * ept

    for p in range(2):
      g = c * 2 + p
      pltpu.sync_copy(zeros_hbm.at[pl.ds(r0, zr)], acc.at[pl.ds(r0, zr)])
      plsc.subcore_barrier()

      def blk(b, carry):
        e0 = pl.multiple_of(tile_base + b * _K, 8)
        pltpu.sync_copy(src_hbm.at[pl.ds(e0, _K)], sidx)
        pltpu.sync_copy(dst_hbm.at[pl.ds(e0, _K)], didx)

        def jl(j, carry2):
          sv = sidx[pl.ds(j * 16, 16)]
          gidx[pl.ds(j * 16, 16)] = sv * 4 + g
          return carry2

        lax.fori_loop(0, _K // 16, jl, 0)
        pltpu.async_copy(table_hbm.at[gidx], rows, sem).wait()
        pltpu.sync_copy(rows, acc.at[didx], add=True)
        return carry

      lax.fori_loop(0, nblk, blk, 0)
      plsc.subcore_barrier()
      lane0 = pl.multiple_of(g * 16, 16)
      pltpu.sync_copy(acc.at[pl.ds(r0, zr)],
                      out_hbm.at[pl.ds(r0, zr), pl.ds(lane0, 16)])
      plsc.subcore_barrier()

  return kern(table, src, dst, zeros)


def _mlp_a_dense(agg, h, wa, wh, b, bn=4000):
  n = h.shape[0]
  ka = agg.shape[1]
  kh = h.shape[1]
  grid = n // bn

  def kern(agg_ref, h_ref, wa_ref, wh_ref, b_ref, z_ref, st_ref, acc_ref):
    i = pl.program_id(0)
    z = jnp.dot(agg_ref[...], wa_ref[...], preferred_element_type=jnp.float32, precision=lax.Precision.HIGHEST)
    z += jnp.dot(h_ref[...], wh_ref[...], preferred_element_type=jnp.float32, precision=lax.Precision.HIGHEST)
    z += b_ref[...]
    z_ref[...] = z

    @pl.when(i == 0)
    def _():
      acc_ref[...] = jnp.zeros_like(acc_ref)

    acc_ref[0:1, :] += jnp.sum(z, axis=0, keepdims=True)
    acc_ref[1:2, :] += jnp.sum(z * z, axis=0, keepdims=True)

    @pl.when(i == grid - 1)
    def _():
      st_ref[...] = acc_ref[...]

  return pl.pallas_call(
      kern,
      grid=(grid,),
      in_specs=[
          pl.BlockSpec((bn, ka), lambda i: (i, 0)),
          pl.BlockSpec((bn, kh), lambda i: (i, 0)),
          pl.BlockSpec((ka, 64), lambda i: (0, 0)),
          pl.BlockSpec((kh, 64), lambda i: (0, 0)),
          pl.BlockSpec((1, 64), lambda i: (0, 0)),
      ],
      out_specs=[
          pl.BlockSpec((bn, 64), lambda i: (i, 0)),
          pl.BlockSpec((8, 64), lambda i: (0, 0)),
      ],
      out_shape=[
          jax.ShapeDtypeStruct((n, 64), jnp.float32),
          jax.ShapeDtypeStruct((8, 64), jnp.float32),
      ],
      scratch_shapes=[pltpu.VMEM((8, 64), jnp.float32)],
      compiler_params=pltpu.CompilerParams(
          dimension_semantics=("arbitrary",)),
  )(agg, h, wa, wh, b)


def _mlp_b(z, stats, gamma, beta, w2, b2, bn=4000):
  n = z.shape[0]
  grid = n // bn
  inv_n = 1.0 / n

  def kern(z_ref, st_ref, g_ref, be_ref, w2_ref, b2_ref, h_ref):
    m = st_ref[0:1, :] * inv_n
    v = st_ref[1:2, :] * inv_n - m * m
    scale = lax.rsqrt(v + 1e-5) * g_ref[...]
    zz = (z_ref[...] - m) * scale + be_ref[...]
    zz = jnp.maximum(zz, 0.0)
    hh = jnp.dot(zz, w2_ref[...], preferred_element_type=jnp.float32, precision=lax.Precision.HIGHEST)
    h_ref[...] = jnp.maximum(hh + b2_ref[...], 0.0)

  return pl.pallas_call(
      kern,
      grid=(grid,),
      in_specs=[
          pl.BlockSpec((bn, 64), lambda i: (i, 0)),
          pl.BlockSpec((8, 64), lambda i: (0, 0)),
          pl.BlockSpec((1, 64), lambda i: (0, 0)),
          pl.BlockSpec((1, 64), lambda i: (0, 0)),
          pl.BlockSpec((64, 64), lambda i: (0, 0)),
          pl.BlockSpec((1, 64), lambda i: (0, 0)),
      ],
      out_specs=pl.BlockSpec((bn, 64), lambda i: (i, 0)),
      out_shape=jax.ShapeDtypeStruct((n, 64), jnp.float32),
  )(z, stats, gamma, beta, w2, b2)


def _mlp_b_pool_head(z, stats, gamma, beta, w2, b2, batch3, l1w, l1b, l2w,
                     l2b, bn=2000):
  n = z.shape[0]
  grid = n // bn
  inv_n = 1.0 / n

  def kern(z_ref, st_ref, g_ref, be_ref, w2_ref, b2_ref, batch_ref,
           l1w_ref, l1b_ref, l2w_ref, l2b_ref, out_ref, pool_ref):
    i = pl.program_id(0)
    m = st_ref[0:1, :] * inv_n
    v = st_ref[1:2, :] * inv_n - m * m
    scale = lax.rsqrt(v + 1e-5) * g_ref[...]
    zz = (z_ref[...] - m) * scale + be_ref[...]
    zz = jnp.maximum(zz, 0.0)
    hh = jnp.dot(zz, w2_ref[...], preferred_element_type=jnp.float32, precision=lax.Precision.HIGHEST)
    hh = jnp.maximum(hh + b2_ref[...], 0.0)

    seg = batch_ref[0, 0]
    onehot = (seg[:, None] ==
              lax.broadcasted_iota(jnp.int32, (bn, _NG), 1)).astype(
                  jnp.float32)
    contrib = lax.dot_general(onehot, hh, (((0,), (0,)), ((), ())),
                              preferred_element_type=jnp.float32,
                              precision=lax.Precision.HIGHEST)

    @pl.when(i == 0)
    def _():
      pool_ref[...] = jnp.zeros_like(pool_ref)

    pool_ref[...] += contrib

    @pl.when(i == grid - 1)
    def _():
      p = pool_ref[...]
      q = jnp.dot(p, l1w_ref[...], preferred_element_type=jnp.float32, precision=lax.Precision.HIGHEST)
      q = jnp.maximum(q + l1b_ref[...], 0.0)
      out_ref[...] = jnp.dot(q, l2w_ref[...],
                             preferred_element_type=jnp.float32, precision=lax.Precision.HIGHEST) + l2b_ref[...]

  return pl.pallas_call(
      kern,
      grid=(grid,),
      in_specs=[
          pl.BlockSpec((bn, 64), lambda i: (i, 0)),
          pl.BlockSpec((8, 64), lambda i: (0, 0)),
          pl.BlockSpec((1, 64), lambda i: (0, 0)),
          pl.BlockSpec((1, 64), lambda i: (0, 0)),
          pl.BlockSpec((64, 64), lambda i: (0, 0)),
          pl.BlockSpec((1, 64), lambda i: (0, 0)),
          pl.BlockSpec((1, 1, bn), lambda i: (i, 0, 0)),
          pl.BlockSpec((64, 64), lambda i: (0, 0)),
          pl.BlockSpec((1, 64), lambda i: (0, 0)),
          pl.BlockSpec((64, 1), lambda i: (0, 0)),
          pl.BlockSpec((1, 1), lambda i: (0, 0)),
      ],
      out_specs=pl.BlockSpec((_NG, 1), lambda i: (0, 0)),
      out_shape=jax.ShapeDtypeStruct((_NG, 1), jnp.float32),
      scratch_shapes=[pltpu.VMEM((_NG, 64), jnp.float32)],
      compiler_params=pltpu.CompilerParams(
          dimension_semantics=("arbitrary",)),
  )(z, stats, gamma, beta, w2, b2, batch3, l1w, l1b, l2w, l2b)


def kernel(x, edge_index, batch,
           c1_W1, c1_b1, c1_g, c1_be, c1_W2, c1_b2,
           c2_W1, c2_b1, c2_g, c2_be, c2_W2, c2_b2,
           c3_W1, c3_b1, c3_g, c3_be, c3_W2, c3_b2,
           lin1_W, lin1_b, lin2_W, lin2_b):
  n = x.shape[0]
  src, dst = edge_index[0], edge_index[1]
  zeros = jnp.zeros((n, 16), jnp.float32)

  x16 = jnp.pad(x, ((0, 0), (0, 16 - x.shape[1])))
  w1p = jnp.pad(c1_W1, ((0, 16 - c1_W1.shape[0]), (0, 0)))
  agg1 = _agg_first_layer(x16, src, dst, zeros)
  z1, st1 = _mlp_a_dense(agg1, x16,
                         jnp.concatenate([w1p, w1p], axis=0), w1p,
                         c1_b1.reshape(1, 64))
  h1 = _mlp_b(z1, st1, c1_g.reshape(1, 64), c1_be.reshape(1, 64), c1_W2,
              c1_b2.reshape(1, 64))

  agg2 = _agg_grouped(h1.reshape(n * 4, 16), src, dst, zeros)
  z2, st2 = _mlp_a_dense(agg2, h1, c2_W1, c2_W1, c2_b1.reshape(1, 64))
  h2 = _mlp_b(z2, st2, c2_g.reshape(1, 64), c2_be.reshape(1, 64), c2_W2,
              c2_b2.reshape(1, 64))

  agg3 = _agg_grouped(h2.reshape(n * 4, 16), src, dst, zeros)
  z3, st3 = _mlp_a_dense(agg3, h2, c3_W1, c3_W1, c3_b1.reshape(1, 64))
  bn_pool = 2000
  batch3 = batch.reshape(n // bn_pool, 1, bn_pool)
  out = _mlp_b_pool_head(z3, st3, c3_g.reshape(1, 64), c3_be.reshape(1, 64),
                         c3_W2, c3_b2.reshape(1, 64), batch3,
                         lin1_W, lin1_b.reshape(1, 64), lin2_W,
                         lin2_b.reshape(1, 1), bn=bn_pool)
  return out
